# baseline (speedup 1.0000x reference)
import jax
import jax.numpy as jnp
from jax.experimental import pallas as pl
from jax.experimental.pallas import tpu as pltpu

_N = 100000
_H = 512
_D = 16
_TN = 2000
_T = _N // _TN
_CT = 0


def _hgnn_body(adj_ref, emb_ref, out_ref, cache, lat):
    p = pl.program_id(0)
    i = pl.program_id(1)

    @pl.when(p == 0)
    def _phase_a():
        @pl.when(i == 0)
        def _():
            lat[...] = jnp.zeros_like(lat)

        a = adj_ref[...]
        e = emb_ref[...]
        lat[...] += jax.lax.dot_general(
            a, e, (((0,), (0,)), ((), ())),
            preferred_element_type=jnp.float32)

        @pl.when(i < _CT)
        def _():
            cache[pl.ds(i * _TN, _TN), :] = a.astype(jnp.bfloat16)

    @pl.when(p == 1)
    def _phase_b():
        lb = lat[...].astype(jnp.bfloat16)

        @pl.when(i < _CT)
        def _cached():
            c = cache[pl.ds(i * _TN, _TN), :]
            out_ref[...] = jax.lax.dot_general(
                c, lb, (((1,), (0,)), ((), ())),
                preferred_element_type=jnp.float32)

        @pl.when(i >= _CT)
        def _streamed():
            a = adj_ref[...].astype(jnp.bfloat16)
            out_ref[...] = jax.lax.dot_general(
                a, lb, (((1,), (0,)), ((), ())),
                preferred_element_type=jnp.float32)


def kernel(adj, embeds):
    return pl.pallas_call(
        _hgnn_body,
        grid=(2, _T),
        in_specs=[
            pl.BlockSpec(
                (_TN, _H),
                lambda p, i: (jnp.where(p == 0, i, jnp.where(i < _CT, _T - 1, i)), 0)),
            pl.BlockSpec((_TN, _D), lambda p, i: (jnp.where(p == 0, i, 0), 0)),
        ],
        out_specs=pl.BlockSpec((_TN, _D), lambda p, i: (jnp.where(p == 0, 0, i), 0)),
        out_shape=jax.ShapeDtypeStruct((_N, _D), jnp.float32),
        scratch_shapes=[
            pltpu.VMEM((max(_CT, 1) * _TN, _H), jnp.bfloat16),
            pltpu.VMEM((_H, _D), jnp.float32),
        ],
        compiler_params=pltpu.CompilerParams(
            dimension_semantics=("arbitrary", "arbitrary"),
            vmem_limit_bytes=64 * 1024 * 1024,
        ),
    )(adj, embeds)


# speedup vs baseline: 1.0789x; 1.0789x over previous
import jax
import jax.numpy as jnp
from jax.experimental import pallas as pl
from jax.experimental.pallas import tpu as pltpu

_N = 100000
_H = 512
_D = 16
_TN = 5000
_T = _N // _TN
_CT = 0


def _hgnn_body(adj_ref, emb_ref, out_ref, cache, lat):
    p = pl.program_id(0)
    i = pl.program_id(1)

    @pl.when(p == 0)
    def _phase_a():
        @pl.when(i == 0)
        def _():
            lat[...] = jnp.zeros_like(lat)

        a = adj_ref[...]
        e = emb_ref[...]
        lat[...] += jax.lax.dot_general(
            a, e, (((0,), (0,)), ((), ())),
            preferred_element_type=jnp.float32)

        @pl.when(i < _CT)
        def _():
            cache[pl.ds(i * _TN, _TN), :] = a.astype(jnp.bfloat16)

    @pl.when(p == 1)
    def _phase_b():
        lb = lat[...].astype(jnp.bfloat16)

        @pl.when(i < _CT)
        def _cached():
            c = cache[pl.ds(i * _TN, _TN), :]
            out_ref[...] = jax.lax.dot_general(
                c, lb, (((1,), (0,)), ((), ())),
                preferred_element_type=jnp.float32)

        @pl.when(i >= _CT)
        def _streamed():
            a = adj_ref[...].astype(jnp.bfloat16)
            out_ref[...] = jax.lax.dot_general(
                a, lb, (((1,), (0,)), ((), ())),
                preferred_element_type=jnp.float32)


def kernel(adj, embeds):
    return pl.pallas_call(
        _hgnn_body,
        grid=(2, _T),
        in_specs=[
            pl.BlockSpec(
                (_TN, _H),
                lambda p, i: (jnp.where(p == 0, i, jnp.where(i < _CT, _T - 1, i)), 0)),
            pl.BlockSpec((_TN, _D), lambda p, i: (jnp.where(p == 0, i, 0), 0)),
        ],
        out_specs=pl.BlockSpec((_TN, _D), lambda p, i: (jnp.where(p == 0, 0, i), 0)),
        out_shape=jax.ShapeDtypeStruct((_N, _D), jnp.float32),
        scratch_shapes=[
            pltpu.VMEM((max(_CT, 1) * _TN, _H), jnp.bfloat16),
            pltpu.VMEM((_H, _D), jnp.float32),
        ],
        compiler_params=pltpu.CompilerParams(
            dimension_semantics=("arbitrary", "arbitrary"),
            vmem_limit_bytes=64 * 1024 * 1024,
        ),
    )(adj, embeds)


# speedup vs baseline: 1.0879x; 1.0083x over previous
import jax
import jax.numpy as jnp
from jax.experimental import pallas as pl
from jax.experimental.pallas import tpu as pltpu

_N = 100000
_H = 512
_D = 16
_TN = 5000
_T = _N // _TN
_CT = 0


def _hgnn_body(adj_ref, emb_ref, out_ref, cache, lat):
    p = pl.program_id(0)
    i = pl.program_id(1)

    @pl.when(p == 0)
    def _phase_a():
        @pl.when(i == 0)
        def _():
            lat[...] = jnp.zeros_like(lat)

        a = adj_ref[...]
        e = emb_ref[...]
        lat[...] += jax.lax.dot_general(
            e, a, (((0,), (0,)), ((), ())),
            preferred_element_type=jnp.float32)

        @pl.when(i < _CT)
        def _():
            cache[pl.ds(i * _TN, _TN), :] = a.astype(jnp.bfloat16)

    @pl.when(p == 1)
    def _phase_b():
        lb = lat[...].astype(jnp.bfloat16)

        @pl.when(i < _CT)
        def _cached():
            c = cache[pl.ds(i * _TN, _TN), :]
            out_ref[...] = jax.lax.dot_general(
                c, lb, (((1,), (1,)), ((), ())),
                preferred_element_type=jnp.float32)

        @pl.when(i >= _CT)
        def _streamed():
            a = adj_ref[...].astype(jnp.bfloat16)
            out_ref[...] = jax.lax.dot_general(
                a, lb, (((1,), (1,)), ((), ())),
                preferred_element_type=jnp.float32)


def kernel(adj, embeds):
    return pl.pallas_call(
        _hgnn_body,
        grid=(2, _T),
        in_specs=[
            pl.BlockSpec(
                (_TN, _H),
                lambda p, i: (jnp.where(p == 0, i, jnp.where(i < _CT, _T - 1, i)), 0)),
            pl.BlockSpec((_TN, _D), lambda p, i: (jnp.where(p == 0, i, 0), 0)),
        ],
        out_specs=pl.BlockSpec((_TN, _D), lambda p, i: (jnp.where(p == 0, 0, i), 0)),
        out_shape=jax.ShapeDtypeStruct((_N, _D), jnp.float32),
        scratch_shapes=[
            pltpu.VMEM((max(_CT, 1) * _TN, _H), jnp.bfloat16),
            pltpu.VMEM((_D, _H), jnp.float32),
        ],
        compiler_params=pltpu.CompilerParams(
            dimension_semantics=("arbitrary", "arbitrary"),
            vmem_limit_bytes=64 * 1024 * 1024,
        ),
    )(adj, embeds)


# speedup vs baseline: 1.0966x; 1.0080x over previous
import jax
import jax.numpy as jnp
from jax.experimental import pallas as pl
from jax.experimental.pallas import tpu as pltpu

_N = 100000
_H = 512
_HH = _H // 2
_D = 16
_TN = 5000
_T = _N // _TN
_CT = 0


def _hgnn_body(adjl_ref, adjr_ref, emb_ref, out_ref, cachel, cacher, lat):
    p = pl.program_id(0)
    i = pl.program_id(1)

    @pl.when(p == 0)
    def _phase_a():
        @pl.when(i == 0)
        def _():
            lat[...] = jnp.zeros_like(lat)

        al = adjl_ref[...]
        ar = adjr_ref[...]
        e = emb_ref[...]
        lat[:, :_HH] += jax.lax.dot_general(
            e, al, (((0,), (0,)), ((), ())),
            preferred_element_type=jnp.float32)
        lat[:, _HH:] += jax.lax.dot_general(
            e, ar, (((0,), (0,)), ((), ())),
            preferred_element_type=jnp.float32)

        @pl.when(i < _CT)
        def _():
            cachel[pl.ds(i * _TN, _TN), :] = al.astype(jnp.bfloat16)
            cacher[pl.ds(i * _TN, _TN), :] = ar.astype(jnp.bfloat16)

    @pl.when(p == 1)
    def _phase_b():
        lbl = lat[:, :_HH].astype(jnp.bfloat16)
        lbr = lat[:, _HH:].astype(jnp.bfloat16)

        @pl.when(i < _CT)
        def _cached():
            cl = cachel[pl.ds(i * _TN, _TN), :]
            cr = cacher[pl.ds(i * _TN, _TN), :]
            out_ref[...] = (
                jax.lax.dot_general(cl, lbl, (((1,), (1,)), ((), ())),
                                    preferred_element_type=jnp.float32)
                + jax.lax.dot_general(cr, lbr, (((1,), (1,)), ((), ())),
                                      preferred_element_type=jnp.float32))

        @pl.when(i >= _CT)
        def _streamed():
            al = adjl_ref[...].astype(jnp.bfloat16)
            ar = adjr_ref[...].astype(jnp.bfloat16)
            out_ref[...] = (
                jax.lax.dot_general(al, lbl, (((1,), (1,)), ((), ())),
                                    preferred_element_type=jnp.float32)
                + jax.lax.dot_general(ar, lbr, (((1,), (1,)), ((), ())),
                                      preferred_element_type=jnp.float32))


def _adj_index(p, i):
    return jnp.where(p == 0, i, jnp.where(i < _CT, _T - 1, i))


def kernel(adj, embeds):
    return pl.pallas_call(
        _hgnn_body,
        grid=(2, _T),
        in_specs=[
            pl.BlockSpec((_TN, _HH), lambda p, i: (_adj_index(p, i), 0)),
            pl.BlockSpec((_TN, _HH), lambda p, i: (_adj_index(p, i), 1)),
            pl.BlockSpec((_TN, _D), lambda p, i: (jnp.where(p == 0, i, 0), 0)),
        ],
        out_specs=pl.BlockSpec((_TN, _D), lambda p, i: (jnp.where(p == 0, 0, i), 0)),
        out_shape=jax.ShapeDtypeStruct((_N, _D), jnp.float32),
        scratch_shapes=[
            pltpu.VMEM((max(_CT, 1) * _TN, _HH), jnp.bfloat16),
            pltpu.VMEM((max(_CT, 1) * _TN, _HH), jnp.bfloat16),
            pltpu.VMEM((_D, _H), jnp.float32),
        ],
        compiler_params=pltpu.CompilerParams(
            dimension_semantics=("arbitrary", "arbitrary"),
            vmem_limit_bytes=64 * 1024 * 1024,
        ),
    )(adj, adj, embeds)
